# baseline (speedup 1.0000x reference)
import functools

import jax
import jax.numpy as jnp
from jax import lax
from jax.experimental import pallas as pl
from jax.experimental.pallas import tpu as pltpu
from jax.experimental.pallas import tpu_sc as plsc

_N = 10000
_E = 320000
_C = 128
_ACT = 8
_H = 32
_NSC = 2
_NSUB = 16
_NW = _NSC * _NSUB
_CH = 128
_NCHT = 80
_NCH = _NW * _NCHT
_EPAD = _NCH * _CH
_NPAD = _N + 16
_RB = 1000
_G = _N // _RB

_f32 = jnp.float32


def _zslice(s):
    return pl.ds(s * 624, 640)


def _fill(dcur, dst_v, j):
    for i in range(_CH // 16):
        dcur[pl.ds(i * 16, 16)] = dst_v[j, pl.ds(i * 16, 16)]


def _deg_sc_body(dst2, ones2, zeros2, out, dst_v, d0, d1, d2, d3, rows_v, acc,
                 s0, s1, s2, s3):
    c = lax.axis_index("c")
    s = lax.axis_index("s")
    pltpu.sync_copy(zeros2.at[_zslice(s)], acc.at[_zslice(s)])
    pltpu.sync_copy(ones2, rows_v)
    ch0 = (c * _NSUB + s) * _NCHT
    pltpu.sync_copy(dst2.at[pl.ds(ch0, _NCHT)], dst_v)
    plsc.subcore_barrier()

    dcs = (d0, d1, d2, d3)
    sems = (s0, s1, s2, s3)
    for b in range(4):
        _fill(dcs[b], dst_v, b)
        pltpu.async_copy(rows_v, acc.at[dcs[b]], sems[b], add=True)

    def body(i, carry):
        for b in range(4):
            j = 4 * i + b
            pltpu.make_async_copy(rows_v, acc.at[dcs[b]], sems[b]).wait()
            _fill(dcs[b], dst_v, j)
            pltpu.async_copy(rows_v, acc.at[dcs[b]], sems[b], add=True)
        return carry

    lax.fori_loop(1, _NCHT // 4, body, 0)
    for b in range(4):
        pltpu.make_async_copy(rows_v, acc.at[dcs[b]], sems[b]).wait()
    plsc.subcore_barrier()
    pltpu.sync_copy(acc.at[_zslice(s)], out.at[c].at[_zslice(s)])


_PCH = _NCHT // 2


def _scatter_sc_body(src2, dst2, y, zeros2, out, src_v, dst_v,
                     d0, d1, r0, r1, acc, g0, g1, s0, s1):
    c = lax.axis_index("c")
    s = lax.axis_index("s")
    pltpu.sync_copy(zeros2.at[_zslice(s)], acc.at[_zslice(s)])
    ch0 = (c * _NSUB + s) * _NCHT
    plsc.subcore_barrier()

    def startg(j, rbuf, sem):
        pltpu.async_copy(y.at[src_v.at[j]], rbuf, sem)

    def waitg(j, rbuf, sem):
        pltpu.make_async_copy(y.at[src_v.at[j]], rbuf, sem).wait()

    def starts(dc, rbuf, sem):
        pltpu.async_copy(rbuf, acc.at[dc], sem, add=True)

    def waits(dc, rbuf, sem):
        pltpu.make_async_copy(rbuf, acc.at[dc], sem).wait()

    def proc(j, dc, rbuf, gsem, ssem):
        waitg(j, rbuf, gsem)
        _fill(dc, dst_v, j)
        starts(dc, rbuf, ssem)

    for p in range(2):
        base = ch0 + p * _PCH
        pltpu.sync_copy(src2.at[pl.ds(base, _PCH)], src_v)
        pltpu.sync_copy(dst2.at[pl.ds(base, _PCH)], dst_v)
        startg(0, r0, g0)
        startg(1, r1, g1)

        def body(k, carry):
            j = 2 * k
            proc(j, d0, r0, g0, s0)
            proc(j + 1, d1, r1, g1, s1)
            waits(d0, r0, s0)
            startg(j + 2, r0, g0)
            waits(d1, r1, s1)
            startg(j + 3, r1, g1)
            return carry

        lax.fori_loop(0, _PCH // 2 - 1, body, 0)
        proc(_PCH - 2, d0, r0, g0, s0)
        proc(_PCH - 1, d1, r1, g1, s1)
        waits(d0, r0, s0)
        waits(d1, r1, s1)

    plsc.subcore_barrier()
    pltpu.sync_copy(acc.at[_zslice(s)], out.at[c].at[_zslice(s)])


_sc_cache = {}


def _sc_mesh():
    return plsc.VectorSubcoreMesh(core_axis_name="c", subcore_axis_name="s",
                                  num_cores=_NSC, num_subcores=_NSUB)


def _deg_call(dst2, ones2, zeros2):
    if "deg" not in _sc_cache:
        _sc_cache["deg"] = pl.kernel(
            _deg_sc_body,
            out_type=jax.ShapeDtypeStruct((_NSC, _N, _C), _f32),
            mesh=_sc_mesh(),
            scratch_types=[
                pltpu.VMEM((_NCHT, _CH), jnp.int32),
                pltpu.VMEM((_CH,), jnp.int32),
                pltpu.VMEM((_CH,), jnp.int32),
                pltpu.VMEM((_CH,), jnp.int32),
                pltpu.VMEM((_CH,), jnp.int32),
                pltpu.VMEM((_CH, _C), _f32),
                pltpu.VMEM_SHARED((_NPAD, _C), _f32),
                pltpu.SemaphoreType.DMA,
                pltpu.SemaphoreType.DMA,
                pltpu.SemaphoreType.DMA,
                pltpu.SemaphoreType.DMA,
            ],
        )
    return _sc_cache["deg"](dst2, ones2, zeros2)


def _scatter_call(src2, dst2, y, zeros2):
    if "scat" not in _sc_cache:
        _sc_cache["scat"] = pl.kernel(
            _scatter_sc_body,
            out_type=jax.ShapeDtypeStruct((_NSC, _N, _C), _f32),
            mesh=_sc_mesh(),
            scratch_types=(
                [pltpu.VMEM((_PCH, _CH), jnp.int32)] * 2
                + [pltpu.VMEM((_CH,), jnp.int32)] * 2
                + [pltpu.VMEM((_CH, _C), _f32)] * 2
                + [pltpu.VMEM_SHARED((_NPAD, _C), _f32)]
                + [pltpu.SemaphoreType.DMA] * 4
            ),
        )
    return _sc_cache["scat"](src2, dst2, y, zeros2)


def _prologue_tc(degp_ref, state_ref, w1_ref, dis_ref, y1_ref):
    deg = degp_ref[0, :, 0:1] + degp_ref[1, :, 0:1] + 1.0
    dis = lax.rsqrt(deg)
    dis_ref[...] = dis
    y1_ref[...] = dis * jnp.dot(state_ref[...], w1_ref[...],
                                preferred_element_type=_f32)


def _layer_tc(z_ref, y_ref, dis_ref, b_ref, w_ref, out_ref, ynext_ref):
    dis = dis_ref[...]
    o = jnp.maximum(dis * (z_ref[0] + z_ref[1] + y_ref[...]) + b_ref[...], 0.0)
    out_ref[...] = o
    ynext_ref[...] = dis * jnp.dot(o, w_ref[...], preferred_element_type=_f32)


def _final_tc(z_ref, y5_ref, dis_ref, b3_ref, o1_ref, o2_ref, o3_ref, o4_ref,
              st_ref, act_ref, a6_ref, a7_ref, l1b_ref, w2t_ref, l2b_ref,
              w3_ref, l3b_ref, s_ref, out_ref):
    dis = dis_ref[...]
    o5 = jnp.maximum(dis * (z_ref[0] + z_ref[1] + y5_ref[...]) + b3_ref[...],
                     0.0)
    parts = (o1_ref[...], o2_ref[...], o3_ref[...], o4_ref[...], o5,
             st_ref[...])
    acc = l1b_ref[...] + act_ref[...] * a7_ref[...]
    for k in range(6):
        acc = acc + jnp.dot(parts[k], a6_ref[k], preferred_element_type=_f32)
    h1 = jnp.maximum(acc, 0.0)
    h2 = jnp.maximum(
        jnp.dot(h1, w2t_ref[...], preferred_element_type=_f32) + l2b_ref[...],
        0.0)
    g = jnp.dot(s_ref[...], h2, preferred_element_type=_f32,
                precision=lax.Precision.HIGHEST)
    i = pl.program_id(0)
    out_ref[pl.ds(i * (_RB // _ACT), _RB // _ACT), :] = (
        jnp.dot(g, w3_ref[...], preferred_element_type=_f32) + l3b_ref[...])


def _row_spec(i_map=None, shape=(_RB, _C)):
    return pl.BlockSpec(shape, i_map or (lambda i: (i, 0)))


_FULL = lambda shape: pl.BlockSpec(shape, lambda i: tuple(0 for _ in shape))


def _prologue_call(degp, state, w1):
    return pl.pallas_call(
        _prologue_tc,
        grid=(_G,),
        in_specs=[pl.BlockSpec((_NSC, _RB, _C), lambda i: (0, i, 0)),
                  _row_spec(), _FULL((_C, _C))],
        out_specs=[_row_spec(shape=(_RB, 1)), _row_spec()],
        out_shape=[jax.ShapeDtypeStruct((_N, 1), _f32),
                   jax.ShapeDtypeStruct((_N, _C), _f32)],
    )(degp, state, w1)


def _layer_call(z, y, dis, b, w):
    return pl.pallas_call(
        _layer_tc,
        grid=(_G,),
        in_specs=[pl.BlockSpec((_NSC, _RB, _C), lambda i: (0, i, 0)),
                  _row_spec(), _row_spec(shape=(_RB, 1)),
                  _FULL((1, _C)), _FULL((_C, _C))],
        out_specs=[_row_spec(), _row_spec()],
        out_shape=[jax.ShapeDtypeStruct((_N, _C), _f32),
                   jax.ShapeDtypeStruct((_N, _C), _f32)],
    )(z, y, dis, b.reshape(1, _C), w)


def _final_call(z5, y5, dis, b3, o1, o2, o3, o4, state, act_col, a6, a7,
                l1b, w2t, l2b, w3, l3b, smat):
    return pl.pallas_call(
        _final_tc,
        grid=(_G,),
        in_specs=[pl.BlockSpec((_NSC, _RB, _C), lambda i: (0, i, 0)),
                  _row_spec(), _row_spec(shape=(_RB, 1)), _FULL((1, _C)),
                  _row_spec(), _row_spec(), _row_spec(), _row_spec(),
                  _row_spec(), _row_spec(shape=(_RB, 1)),
                  _FULL((6, _C, _H)), _FULL((1, _H)), _FULL((1, _H)),
                  _FULL((_H, _H)), _FULL((1, _H)), _FULL((_H, 1)),
                  _FULL((1, 1)),
                  pl.BlockSpec((_RB // _ACT, _RB), lambda i: (0, 0))],
        out_specs=pl.BlockSpec((_N // _ACT, 1), lambda i: (0, 0)),
        out_shape=jax.ShapeDtypeStruct((_N // _ACT, 1), _f32),
    )(z5, y5, dis, b3.reshape(1, _C), o1, o2, o3, o4, state, act_col, a6, a7,
      l1b.reshape(1, _H), w2t, l2b.reshape(1, _H), w3, l3b.reshape(1, 1), smat)


def kernel(state, edge_index, action, W1, b1, W2, b2, W3, b3,
           lin1W, lin1b, lin2W, lin2b, lin3W, lin3b):
    pad = _EPAD - _E
    padi = jnp.arange(pad, dtype=jnp.int32)
    src2 = jnp.concatenate([edge_index[0], padi % _N]).reshape(_NCH, _CH)
    dst2 = jnp.concatenate([edge_index[1], _N + (padi % 16)]).reshape(_NCH, _CH)
    zeros2 = jnp.zeros((_N, _C), _f32)
    ones2 = jnp.ones((_CH, _C), _f32)

    degp = _deg_call(dst2, ones2, zeros2)
    dis, y1 = _prologue_call(degp, state, W1)

    z1 = _scatter_call(src2, dst2, y1, zeros2)
    out1, y2 = _layer_call(z1, y1, dis, b1, W2)
    z2 = _scatter_call(src2, dst2, y2, zeros2)
    out2, y3 = _layer_call(z2, y2, dis, b2, W3)
    z3 = _scatter_call(src2, dst2, y3, zeros2)
    out3, y4 = _layer_call(z3, y3, dis, b3, W3)
    z4 = _scatter_call(src2, dst2, y4, zeros2)
    out4, y5 = _layer_call(z4, y4, dis, b3, W3)
    z5 = _scatter_call(src2, dst2, y5, zeros2)

    a6 = lin1W[:, :6 * _C].T.reshape(6, _C, _H)
    a7 = lin1W[:, 6 * _C].reshape(1, _H)
    smat = jnp.kron(jnp.eye(_RB // _ACT, dtype=_f32),
                    jnp.ones((1, _ACT), _f32))
    res = _final_call(z5, y5, dis, b3, out1, out2, out3, out4, state,
                      action.reshape(_N, 1), a6, a7, lin1b, lin2W.T, lin2b,
                      lin3W.T, lin3b, smat)
    return res.reshape(_N // _ACT)


# speedup vs baseline: 1.1570x; 1.1570x over previous
import functools

import jax
import jax.numpy as jnp
from jax import lax
from jax.experimental import pallas as pl
from jax.experimental.pallas import tpu as pltpu
from jax.experimental.pallas import tpu_sc as plsc

_N = 10000
_E = 320000
_C = 128
_ACT = 8
_H = 32
_NSC = 2
_NSUB = 16
_NW = _NSC * _NSUB
_CH = 128
_NCHT = 80
_NCH = _NW * _NCHT
_EPAD = _NCH * _CH
_NPAD = _N + 16
_RB = 1000
_G = _N // _RB

_f32 = jnp.float32


def _zslice(s):
    return pl.ds(s * 624, 640)


def _deg_sc_body(dst2, ones2, zeros2, out, dst_v, rows_v, acc, s0, s1, s2, s3):
    c = lax.axis_index("c")
    s = lax.axis_index("s")
    pltpu.sync_copy(zeros2.at[_zslice(s)], acc.at[_zslice(s)])
    pltpu.sync_copy(ones2, rows_v)
    ch0 = (c * _NSUB + s) * _NCHT
    pltpu.sync_copy(dst2.at[pl.ds(ch0, _NCHT)], dst_v)
    plsc.subcore_barrier()

    sems = (s0, s1, s2, s3)
    for b in range(4):
        pltpu.async_copy(rows_v, acc.at[dst_v.at[b]], sems[b], add=True)

    def body(i, carry):
        for b in range(4):
            j = 4 * i + b
            pltpu.make_async_copy(rows_v, acc.at[dst_v.at[j]], sems[b]).wait()
            pltpu.async_copy(rows_v, acc.at[dst_v.at[j]], sems[b], add=True)
        return carry

    lax.fori_loop(1, _NCHT // 4, body, 0)
    for b in range(4):
        pltpu.make_async_copy(rows_v, acc.at[dst_v.at[b]], sems[b]).wait()
    plsc.subcore_barrier()
    pltpu.sync_copy(acc.at[_zslice(s)], out.at[c].at[_zslice(s)])


_PCH = _NCHT // 2
_SCH = 64
_SNCHT = _EPAD // (_NW * _SCH)
_SPCH = _SNCHT // 4


def _scatter_sc_body(src2, dst2, y, zeros2, out, src_v, dst_v,
                     r0, r1, r2, r3, acc, g0, g1, g2, g3, s0, s1, s2, s3):
    c = lax.axis_index("c")
    s = lax.axis_index("s")
    pltpu.sync_copy(zeros2.at[_zslice(s)], acc.at[_zslice(s)])
    ch0 = (c * _NSUB + s) * _SNCHT
    plsc.subcore_barrier()

    rbs = (r0, r1, r2, r3)
    gss = (g0, g1, g2, g3)
    sss = (s0, s1, s2, s3)

    def startg(j, rbuf, sem):
        pltpu.async_copy(y.at[src_v.at[j]], rbuf, sem)

    def waitg(j, rbuf, sem):
        pltpu.make_async_copy(y.at[src_v.at[j]], rbuf, sem).wait()

    def starts(j, rbuf, sem):
        pltpu.async_copy(rbuf, acc.at[dst_v.at[j]], sem, add=True)

    def waits(j, rbuf, sem):
        pltpu.make_async_copy(rbuf, acc.at[dst_v.at[j]], sem).wait()

    for p in range(4):
        base = ch0 + p * _SPCH
        pltpu.sync_copy(src2.at[pl.ds(base, _SPCH)], src_v)
        pltpu.sync_copy(dst2.at[pl.ds(base, _SPCH)], dst_v)
        for b in range(4):
            startg(b, rbs[b], gss[b])

        def body(i, carry):
            for b in range(4):
                j = 4 * i + b
                waitg(j, rbs[b], gss[b])
                starts(j, rbs[b], sss[b])
            for b in range(4):
                j = 4 * i + b
                waits(j, rbs[b], sss[b])
                startg(j + 4, rbs[b], gss[b])
            return carry

        lax.fori_loop(0, _SPCH // 4 - 1, body, 0)
        for b in range(4):
            j = _SPCH - 4 + b
            waitg(j, rbs[b], gss[b])
            starts(j, rbs[b], sss[b])
        for b in range(4):
            waits(_SPCH - 4 + b, rbs[b], sss[b])

    plsc.subcore_barrier()
    pltpu.sync_copy(acc.at[_zslice(s)], out.at[c].at[_zslice(s)])


_sc_cache = {}


def _sc_mesh():
    return plsc.VectorSubcoreMesh(core_axis_name="c", subcore_axis_name="s",
                                  num_cores=_NSC, num_subcores=_NSUB)


def _deg_call(dst2, ones2, zeros2):
    if "deg" not in _sc_cache:
        _sc_cache["deg"] = pl.kernel(
            _deg_sc_body,
            out_type=jax.ShapeDtypeStruct((_NSC, _N, _C), _f32),
            mesh=_sc_mesh(),
            scratch_types=[
                pltpu.VMEM((_NCHT, _CH), jnp.int32),
                pltpu.VMEM((_CH, _C), _f32),
                pltpu.VMEM_SHARED((_NPAD, _C), _f32),
                pltpu.SemaphoreType.DMA,
                pltpu.SemaphoreType.DMA,
                pltpu.SemaphoreType.DMA,
                pltpu.SemaphoreType.DMA,
            ],
        )
    return _sc_cache["deg"](dst2, ones2, zeros2)


def _scatter_call(src2, dst2, y, zeros2):
    if "scat" not in _sc_cache:
        _sc_cache["scat"] = pl.kernel(
            _scatter_sc_body,
            out_type=jax.ShapeDtypeStruct((_NSC, _N, _C), _f32),
            mesh=_sc_mesh(),
            scratch_types=(
                [pltpu.VMEM((_SPCH, _SCH), jnp.int32)] * 2
                + [pltpu.VMEM((_SCH, _C), _f32)] * 4
                + [pltpu.VMEM_SHARED((_NPAD, _C), _f32)]
                + [pltpu.SemaphoreType.DMA] * 8
            ),
        )
    return _sc_cache["scat"](src2.reshape(-1, _SCH), dst2.reshape(-1, _SCH),
                             y, zeros2)


def _prologue_tc(degp_ref, state_ref, w1_ref, dis_ref, y1_ref):
    deg = degp_ref[0, :, 0:1] + degp_ref[1, :, 0:1] + 1.0
    dis = lax.rsqrt(deg)
    dis_ref[...] = dis
    y1_ref[...] = dis * jnp.dot(state_ref[...], w1_ref[...],
                                preferred_element_type=_f32)


def _layer_tc(z_ref, y_ref, dis_ref, b_ref, w_ref, out_ref, ynext_ref):
    dis = dis_ref[...]
    o = jnp.maximum(dis * (z_ref[0] + z_ref[1] + y_ref[...]) + b_ref[...], 0.0)
    out_ref[...] = o
    ynext_ref[...] = dis * jnp.dot(o, w_ref[...], preferred_element_type=_f32)


def _final_tc(z_ref, y5_ref, dis_ref, b3_ref, o1_ref, o2_ref, o3_ref, o4_ref,
              st_ref, act_ref, a6_ref, a7_ref, l1b_ref, w2t_ref, l2b_ref,
              w3_ref, l3b_ref, s_ref, out_ref):
    dis = dis_ref[...]
    o5 = jnp.maximum(dis * (z_ref[0] + z_ref[1] + y5_ref[...]) + b3_ref[...],
                     0.0)
    parts = (o1_ref[...], o2_ref[...], o3_ref[...], o4_ref[...], o5,
             st_ref[...])
    acc = l1b_ref[...] + act_ref[...] * a7_ref[...]
    for k in range(6):
        acc = acc + jnp.dot(parts[k], a6_ref[k], preferred_element_type=_f32)
    h1 = jnp.maximum(acc, 0.0)
    h2 = jnp.maximum(
        jnp.dot(h1, w2t_ref[...], preferred_element_type=_f32) + l2b_ref[...],
        0.0)
    g = jnp.dot(s_ref[...], h2, preferred_element_type=_f32,
                precision=lax.Precision.HIGHEST)
    i = pl.program_id(0)
    out_ref[pl.ds(i * (_RB // _ACT), _RB // _ACT), :] = (
        jnp.dot(g, w3_ref[...], preferred_element_type=_f32) + l3b_ref[...])


def _row_spec(i_map=None, shape=(_RB, _C)):
    return pl.BlockSpec(shape, i_map or (lambda i: (i, 0)))


_FULL = lambda shape: pl.BlockSpec(shape, lambda i: tuple(0 for _ in shape))


def _prologue_call(degp, state, w1):
    return pl.pallas_call(
        _prologue_tc,
        grid=(_G,),
        in_specs=[pl.BlockSpec((_NSC, _RB, _C), lambda i: (0, i, 0)),
                  _row_spec(), _FULL((_C, _C))],
        out_specs=[_row_spec(shape=(_RB, 1)), _row_spec()],
        out_shape=[jax.ShapeDtypeStruct((_N, 1), _f32),
                   jax.ShapeDtypeStruct((_N, _C), _f32)],
    )(degp, state, w1)


def _layer_call(z, y, dis, b, w):
    return pl.pallas_call(
        _layer_tc,
        grid=(_G,),
        in_specs=[pl.BlockSpec((_NSC, _RB, _C), lambda i: (0, i, 0)),
                  _row_spec(), _row_spec(shape=(_RB, 1)),
                  _FULL((1, _C)), _FULL((_C, _C))],
        out_specs=[_row_spec(), _row_spec()],
        out_shape=[jax.ShapeDtypeStruct((_N, _C), _f32),
                   jax.ShapeDtypeStruct((_N, _C), _f32)],
    )(z, y, dis, b.reshape(1, _C), w)


def _final_call(z5, y5, dis, b3, o1, o2, o3, o4, state, act_col, a6, a7,
                l1b, w2t, l2b, w3, l3b, smat):
    return pl.pallas_call(
        _final_tc,
        grid=(_G,),
        in_specs=[pl.BlockSpec((_NSC, _RB, _C), lambda i: (0, i, 0)),
                  _row_spec(), _row_spec(shape=(_RB, 1)), _FULL((1, _C)),
                  _row_spec(), _row_spec(), _row_spec(), _row_spec(),
                  _row_spec(), _row_spec(shape=(_RB, 1)),
                  _FULL((6, _C, _H)), _FULL((1, _H)), _FULL((1, _H)),
                  _FULL((_H, _H)), _FULL((1, _H)), _FULL((_H, 1)),
                  _FULL((1, 1)),
                  pl.BlockSpec((_RB // _ACT, _RB), lambda i: (0, 0))],
        out_specs=pl.BlockSpec((_N // _ACT, 1), lambda i: (0, 0)),
        out_shape=jax.ShapeDtypeStruct((_N // _ACT, 1), _f32),
    )(z5, y5, dis, b3.reshape(1, _C), o1, o2, o3, o4, state, act_col, a6, a7,
      l1b.reshape(1, _H), w2t, l2b.reshape(1, _H), w3, l3b.reshape(1, 1), smat)


def kernel(state, edge_index, action, W1, b1, W2, b2, W3, b3,
           lin1W, lin1b, lin2W, lin2b, lin3W, lin3b):
    pad = _EPAD - _E
    padi = jnp.arange(pad, dtype=jnp.int32)
    src2 = jnp.concatenate([edge_index[0], padi % _N]).reshape(_NCH, _CH)
    dst2 = jnp.concatenate([edge_index[1], _N + (padi % 16)]).reshape(_NCH, _CH)
    zeros2 = jnp.zeros((_N, _C), _f32)
    ones2 = jnp.ones((_CH, _C), _f32)

    degp = _deg_call(dst2, ones2, zeros2)
    dis, y1 = _prologue_call(degp, state, W1)

    z1 = _scatter_call(src2, dst2, y1, zeros2)
    out1, y2 = _layer_call(z1, y1, dis, b1, W2)
    z2 = _scatter_call(src2, dst2, y2, zeros2)
    out2, y3 = _layer_call(z2, y2, dis, b2, W3)
    z3 = _scatter_call(src2, dst2, y3, zeros2)
    out3, y4 = _layer_call(z3, y3, dis, b3, W3)
    z4 = _scatter_call(src2, dst2, y4, zeros2)
    out4, y5 = _layer_call(z4, y4, dis, b3, W3)
    z5 = _scatter_call(src2, dst2, y5, zeros2)

    a6 = lin1W[:, :6 * _C].T.reshape(6, _C, _H)
    a7 = lin1W[:, 6 * _C].reshape(1, _H)
    smat = jnp.kron(jnp.eye(_RB // _ACT, dtype=_f32),
                    jnp.ones((1, _ACT), _f32))
    res = _final_call(z5, y5, dis, b3, out1, out2, out3, out4, state,
                      action.reshape(_N, 1), a6, a7, lin1b, lin2W.T, lin2b,
                      lin3W.T, lin3b, smat)
    return res.reshape(_N // _ACT)
